# baseline (speedup 1.0000x reference)
import functools

import jax
import jax.numpy as jnp
from jax import lax
from jax.experimental import pallas as pl
from jax.experimental.pallas import tpu as pltpu
from jax.experimental.pallas import tpu_sc as plsc

EMB = 128
N = 100000
NC, NS = 2, 16
NW = NC * NS
NG = N // 8
GQ, GR = divmod(NG, NW)
CNT_LO = 8 * GQ
CNT_HI = CNT_LO + 8
CH = 128
NFULL = CNT_LO // CH
TAIL = CNT_LO - NFULL * CH
NBUF = 6
NPIPE = NFULL // NBUF

_mesh = plsc.VectorSubcoreMesh(
    core_axis_name="c", subcore_axis_name="s", num_cores=NC, num_subcores=NS
)


@functools.partial(
    pl.kernel,
    out_type=jax.ShapeDtypeStruct((N, EMB), jnp.float32),
    mesh=_mesh,
    scratch_types=[
        pltpu.VMEM((CNT_HI,), jnp.int32),
        [pltpu.VMEM((CH, EMB), jnp.float32) for _ in range(NBUF)],
        [pltpu.SemaphoreType.DMA for _ in range(NBUF)],
        [pltpu.SemaphoreType.DMA for _ in range(NBUF)],
    ],
)
def _emb_lookup(table_hbm, idx_hbm, out_hbm, idx_v, rows, gsems, wsems):
    wid = lax.axis_index("s") * NC + lax.axis_index("c")
    base = 8 * (GQ * wid + jnp.minimum(wid, GR))
    has_extra = wid < GR

    pltpu.sync_copy(
        idx_hbm.at[pl.ds(base, CNT_LO)], idx_v.at[pl.ds(0, CNT_LO)]
    )

    @pl.when(has_extra)
    def _():
        pltpu.sync_copy(
            idx_hbm.at[pl.ds(base + CNT_LO, 8)], idx_v.at[pl.ds(CNT_LO, 8)]
        )

    def fire_g(j, b):
        pltpu.async_copy(
            table_hbm.at[idx_v.at[pl.ds(j * CH, CH)]], rows[b], gsems[b]
        )

    def drain_g(j, b):
        pltpu.make_async_copy(
            table_hbm.at[idx_v.at[pl.ds(j * CH, CH)]], rows[b], gsems[b]
        ).wait()

    def fire_w(j, b):
        pltpu.async_copy(
            rows[b], out_hbm.at[pl.ds(base + j * CH, CH)], wsems[b]
        )

    def drain_w(j, b):
        pltpu.make_async_copy(
            rows[b], out_hbm.at[pl.ds(base + j * CH, CH)], wsems[b]
        ).wait()

    for b in range(NBUF):
        fire_g(b, b)

    def step(p, carry):
        for b in range(NBUF):
            j = p * NBUF + b
            drain_g(j, b)
            fire_w(j, b)
        for b in range(NBUF):
            j = p * NBUF + b

            @pl.when(p < NPIPE - 1)
            def _():
                drain_w(j, b)
                fire_g(j + NBUF, b)

        return carry

    lax.fori_loop(0, NPIPE, step, 0)

    t0 = NFULL * CH
    drain_w((NPIPE - 1) * NBUF + 0, 0)
    pltpu.async_copy(
        table_hbm.at[idx_v.at[pl.ds(t0, TAIL)]],
        rows[0].at[pl.ds(0, TAIL)],
        gsems[0],
    )
    drain_w((NPIPE - 1) * NBUF + 1, 1)

    @pl.when(has_extra)
    def _():
        pltpu.async_copy(
            table_hbm.at[idx_v.at[pl.ds(CNT_LO, 8)]],
            rows[1].at[pl.ds(0, 8)],
            gsems[1],
        )

    for b in range(2, NBUF):
        drain_w((NPIPE - 1) * NBUF + b, b)

    pltpu.make_async_copy(
        table_hbm.at[idx_v.at[pl.ds(t0, TAIL)]],
        rows[0].at[pl.ds(0, TAIL)],
        gsems[0],
    ).wait()
    pltpu.sync_copy(
        rows[0].at[pl.ds(0, TAIL)], out_hbm.at[pl.ds(base + t0, TAIL)]
    )

    @pl.when(has_extra)
    def _():
        pltpu.make_async_copy(
            table_hbm.at[idx_v.at[pl.ds(CNT_LO, 8)]],
            rows[1].at[pl.ds(0, 8)],
            gsems[1],
        ).wait()
        pltpu.sync_copy(
            rows[1].at[pl.ds(0, 8)], out_hbm.at[pl.ds(base + CNT_LO, 8)]
        )


def kernel(Z, W):
    table = jnp.pad(W, ((1, 0), (0, 0)))
    return _emb_lookup(table, Z.astype(jnp.int32))


# speedup vs baseline: 4.0871x; 4.0871x over previous
import functools

import jax
import jax.numpy as jnp
from jax import lax
from jax.experimental import pallas as pl
from jax.experimental.pallas import tpu as pltpu
from jax.experimental.pallas import tpu_sc as plsc

EMB = 128
N = 100000
NUM_ROWS = 65
NC, NS = 2, 16
NW = NC * NS
NG = N // 8
GQ, GR = divmod(NG, NW)
CNT_LO = 8 * GQ
CNT_HI = CNT_LO + 8
CH = 128
NFULL = CNT_LO // CH
TAIL = CNT_LO - NFULL * CH
NBUF = 6
NPIPE = NFULL // NBUF

_mesh = plsc.VectorSubcoreMesh(
    core_axis_name="c", subcore_axis_name="s", num_cores=NC, num_subcores=NS
)


@functools.partial(
    pl.kernel,
    out_type=jax.ShapeDtypeStruct((N, EMB), jnp.float32),
    mesh=_mesh,
    scratch_types=[
        pltpu.VMEM_SHARED((NUM_ROWS, EMB), jnp.float32),
        pltpu.VMEM((CNT_HI,), jnp.int32),
        [pltpu.VMEM((CH, EMB), jnp.float32) for _ in range(NBUF)],
        [pltpu.SemaphoreType.DMA for _ in range(NBUF)],
        [pltpu.SemaphoreType.DMA for _ in range(NBUF)],
    ],
)
def _emb_lookup(table_hbm, idx_hbm, out_hbm, table_sp, idx_v, rows, gsems, wsems):
    wid = lax.axis_index("s") * NC + lax.axis_index("c")
    base = 8 * (GQ * wid + jnp.minimum(wid, GR))
    has_extra = wid < GR

    @pl.when(lax.axis_index("s") == 0)
    def _():
        pltpu.sync_copy(table_hbm, table_sp)

    pltpu.sync_copy(
        idx_hbm.at[pl.ds(base, CNT_LO)], idx_v.at[pl.ds(0, CNT_LO)]
    )

    @pl.when(has_extra)
    def _():
        pltpu.sync_copy(
            idx_hbm.at[pl.ds(base + CNT_LO, 8)], idx_v.at[pl.ds(CNT_LO, 8)]
        )

    plsc.subcore_barrier()

    def fire_g(j, b):
        pltpu.async_copy(
            table_sp.at[idx_v.at[pl.ds(j * CH, CH)]], rows[b], gsems[b]
        )

    def drain_g(j, b):
        pltpu.make_async_copy(
            table_sp.at[idx_v.at[pl.ds(j * CH, CH)]], rows[b], gsems[b]
        ).wait()

    def fire_w(j, b):
        pltpu.async_copy(
            rows[b], out_hbm.at[pl.ds(base + j * CH, CH)], wsems[b]
        )

    def drain_w(j, b):
        pltpu.make_async_copy(
            rows[b], out_hbm.at[pl.ds(base + j * CH, CH)], wsems[b]
        ).wait()

    for b in range(NBUF):
        fire_g(b, b)

    def step(p, carry):
        for b in range(NBUF):
            j = p * NBUF + b
            drain_g(j, b)
            fire_w(j, b)
        for b in range(NBUF):
            j = p * NBUF + b

            @pl.when(p < NPIPE - 1)
            def _():
                drain_w(j, b)
                fire_g(j + NBUF, b)

        return carry

    lax.fori_loop(0, NPIPE, step, 0)

    t0 = NFULL * CH
    drain_w((NPIPE - 1) * NBUF + 0, 0)
    pltpu.async_copy(
        table_sp.at[idx_v.at[pl.ds(t0, TAIL)]],
        rows[0].at[pl.ds(0, TAIL)],
        gsems[0],
    )
    drain_w((NPIPE - 1) * NBUF + 1, 1)

    @pl.when(has_extra)
    def _():
        pltpu.async_copy(
            table_sp.at[idx_v.at[pl.ds(CNT_LO, 8)]],
            rows[1].at[pl.ds(0, 8)],
            gsems[1],
        )

    for b in range(2, NBUF):
        drain_w((NPIPE - 1) * NBUF + b, b)

    pltpu.make_async_copy(
        table_sp.at[idx_v.at[pl.ds(t0, TAIL)]],
        rows[0].at[pl.ds(0, TAIL)],
        gsems[0],
    ).wait()
    pltpu.sync_copy(
        rows[0].at[pl.ds(0, TAIL)], out_hbm.at[pl.ds(base + t0, TAIL)]
    )

    @pl.when(has_extra)
    def _():
        pltpu.make_async_copy(
            table_sp.at[idx_v.at[pl.ds(CNT_LO, 8)]],
            rows[1].at[pl.ds(0, 8)],
            gsems[1],
        ).wait()
        pltpu.sync_copy(
            rows[1].at[pl.ds(0, 8)], out_hbm.at[pl.ds(base + CNT_LO, 8)]
        )


def kernel(Z, W):
    table = jnp.pad(W, ((1, 0), (0, 0)))
    return _emb_lookup(table, Z.astype(jnp.int32))


# speedup vs baseline: 4.1925x; 1.0258x over previous
import functools

import jax
import jax.numpy as jnp
from jax import lax
from jax.experimental import pallas as pl
from jax.experimental.pallas import tpu as pltpu
from jax.experimental.pallas import tpu_sc as plsc

EMB = 128
N = 100000
NUM_ROWS = 65
NC, NS = 2, 16
NW = NC * NS
NG = N // 8
GQ, GR = divmod(NG, NW)
CNT_LO = 8 * GQ
CNT_HI = CNT_LO + 8
CH = 128
NFULL = CNT_LO // CH
TAIL = CNT_LO - NFULL * CH
NBUF = 6
NPIPE = NFULL // NBUF

_mesh = plsc.VectorSubcoreMesh(
    core_axis_name="c", subcore_axis_name="s", num_cores=NC, num_subcores=NS
)


@functools.partial(
    pl.kernel,
    out_type=jax.ShapeDtypeStruct((N, EMB), jnp.float32),
    mesh=_mesh,
    scratch_types=[
        pltpu.VMEM_SHARED((NUM_ROWS, EMB), jnp.float32),
        pltpu.VMEM((CNT_HI,), jnp.int32),
        [pltpu.VMEM((CH, EMB), jnp.float32) for _ in range(NBUF)],
        [pltpu.SemaphoreType.DMA for _ in range(NBUF)],
        [pltpu.SemaphoreType.DMA for _ in range(NBUF)],
    ],
)
def _emb_lookup(table_hbm, idx_hbm, out_hbm, table_sp, idx_v, rows, gsems, wsems):
    wid = lax.axis_index("s") * NC + lax.axis_index("c")
    base = 8 * (GQ * wid + jnp.minimum(wid, GR))
    has_extra = wid < GR

    idx_cp = pltpu.make_async_copy(
        idx_hbm.at[pl.ds(base, CNT_LO)], idx_v.at[pl.ds(0, CNT_LO)], gsems[0]
    )
    idx_cp.start()
    extra_cp = pltpu.make_async_copy(
        idx_hbm.at[pl.ds(base + CNT_LO, 8)], idx_v.at[pl.ds(CNT_LO, 8)], gsems[1]
    )

    @pl.when(has_extra)
    def _():
        extra_cp.start()

    @pl.when(lax.axis_index("s") == 0)
    def _():
        pltpu.sync_copy(table_hbm, table_sp.at[pl.ds(1, NUM_ROWS - 1)])

    idx_cp.wait()

    @pl.when(has_extra)
    def _():
        extra_cp.wait()

    plsc.subcore_barrier()

    def fire_g(j, b):
        pltpu.async_copy(
            table_sp.at[idx_v.at[pl.ds(j * CH, CH)]], rows[b], gsems[b]
        )

    def drain_g(j, b):
        pltpu.make_async_copy(
            table_sp.at[idx_v.at[pl.ds(j * CH, CH)]], rows[b], gsems[b]
        ).wait()

    def fire_w(j, b):
        pltpu.async_copy(
            rows[b], out_hbm.at[pl.ds(base + j * CH, CH)], wsems[b]
        )

    def drain_w(j, b):
        pltpu.make_async_copy(
            rows[b], out_hbm.at[pl.ds(base + j * CH, CH)], wsems[b]
        ).wait()

    for b in range(NBUF):
        fire_g(b, b)

    def step(p, carry):
        for b in range(NBUF):
            j = p * NBUF + b
            drain_g(j, b)
            fire_w(j, b)
        for b in range(NBUF):
            j = p * NBUF + b

            @pl.when(p < NPIPE - 1)
            def _():
                drain_w(j, b)
                fire_g(j + NBUF, b)

        return carry

    lax.fori_loop(0, NPIPE, step, 0)

    t0 = NFULL * CH
    drain_w((NPIPE - 1) * NBUF + 0, 0)
    pltpu.async_copy(
        table_sp.at[idx_v.at[pl.ds(t0, TAIL)]],
        rows[0].at[pl.ds(0, TAIL)],
        gsems[0],
    )
    drain_w((NPIPE - 1) * NBUF + 1, 1)

    @pl.when(has_extra)
    def _():
        pltpu.async_copy(
            table_sp.at[idx_v.at[pl.ds(CNT_LO, 8)]],
            rows[1].at[pl.ds(0, 8)],
            gsems[1],
        )

    for b in range(2, NBUF):
        drain_w((NPIPE - 1) * NBUF + b, b)

    pltpu.make_async_copy(
        table_sp.at[idx_v.at[pl.ds(t0, TAIL)]],
        rows[0].at[pl.ds(0, TAIL)],
        gsems[0],
    ).wait()
    pltpu.sync_copy(
        rows[0].at[pl.ds(0, TAIL)], out_hbm.at[pl.ds(base + t0, TAIL)]
    )

    @pl.when(has_extra)
    def _():
        pltpu.make_async_copy(
            table_sp.at[idx_v.at[pl.ds(CNT_LO, 8)]],
            rows[1].at[pl.ds(0, 8)],
            gsems[1],
        ).wait()
        pltpu.sync_copy(
            rows[1].at[pl.ds(0, 8)], out_hbm.at[pl.ds(base + CNT_LO, 8)]
        )


def kernel(Z, W):
    return _emb_lookup(W, Z.astype(jnp.int32))
